# baseline (speedup 1.0000x reference)
import functools

import jax
import jax.numpy as jnp
from jax import lax
from jax.experimental import pallas as pl
from jax.experimental.pallas import tpu as pltpu
from jax.experimental.pallas import tpu_sc as plsc

VOCAB_ROWS = 1002
D = 50
B, L = 4096, 256
N = B * L
TW = VOCAB_ROWS * D

NUM_CORES = 2
NUM_SUBCORES = 16
NW = NUM_CORES * NUM_SUBCORES
BPW = N // NW

CHUNK = 512
NCHUNK = BPW // CHUNK
IBLK = 8
NIB = NCHUNK // IBLK

_mesh = plsc.VectorSubcoreMesh(core_axis_name="c", subcore_axis_name="s")


@functools.partial(
    pl.kernel,
    mesh=_mesh,
    compiler_params=pltpu.CompilerParams(use_tc_tiling_on_sc=False),
    out_type=jax.ShapeDtypeStruct((N * D,), jnp.float32),
    scratch_types=[
        pltpu.VMEM((IBLK * CHUNK,), jnp.int32),
        pltpu.VMEM((TW,), jnp.float32),
        pltpu.VMEM((CHUNK * D,), jnp.float32),
        pltpu.VMEM((CHUNK * D,), jnp.float32),
        pltpu.SemaphoreType.DMA,
        pltpu.SemaphoreType.DMA,
    ],
)
def _embed_lookup(idx_hbm, table_hbm, out_hbm, idx_v, table_v,
                  packed0, packed1, sem0, sem1):
    wid = lax.axis_index("s") * NUM_CORES + lax.axis_index("c")
    base = wid * BPW
    bufs = (packed0, packed1)
    sems = (sem0, sem1)

    pltpu.sync_copy(table_hbm, table_v)

    def outer(co, carry):
        @pl.when(lax.rem(co, IBLK // 2) == 0)
        def _stage_idx():
            ioff = pl.multiple_of(base + (co * 2) * CHUNK, IBLK * CHUNK)
            pltpu.sync_copy(idx_hbm.at[pl.ds(ioff, IBLK * CHUNK)], idx_v)

        ib = lax.rem(co, IBLK // 2)
        for b in range(2):
            ci = co * 2 + b
            off = pl.multiple_of(base + ci * CHUNK, CHUNK)
            loc = pl.multiple_of((ib * 2 + b) * CHUNK, CHUNK)
            buf = bufs[b]
            sem = sems[b]

            @pl.when(co > 0)
            def _drain():
                pltpu.make_async_copy(
                    buf, out_hbm.at[pl.ds(0, CHUNK * D)], sem
                ).wait()

            def grp_body(g, carry2):
                toks = idx_v[pl.ds(loc + g * 16, 16)]
                for j in range(16):
                    s = toks[j]
                    srcw = s * D
                    dstw = (g * 16 + j) * D
                    for c in (0, 16, 32, 34):
                        buf[pl.ds(dstw + c, 16)] = table_v[pl.ds(srcw + c, 16)]
                return carry2

            lax.fori_loop(0, CHUNK // 16, grp_body, 0, unroll=2)

            pltpu.async_copy(buf, out_hbm.at[pl.ds(off * D, CHUNK * D)], sem)
        return carry

    lax.fori_loop(0, NCHUNK // 2, outer, 0)

    for b in range(2):
        pltpu.make_async_copy(
            bufs[b], out_hbm.at[pl.ds(0, CHUNK * D)], sems[b]
        ).wait()


def kernel(seqs, att_mask, word_embedding):
    del att_mask
    idx = seqs.reshape(N).astype(jnp.int32)
    out = _embed_lookup(idx, word_embedding.reshape(TW))
    return out.reshape(B, L, D)
